# Initial kernel scaffold; baseline (speedup 1.0000x reference)
import jax
import jax.numpy as jnp
from jax.experimental import pallas as pl


def kernel(d_u, p_u, edge_index, Wq, Wk, Wv, e1, e2):
    raise NotImplementedError("write your pallas kernel here")


# speedup vs baseline: 23.5443x; 23.5443x over previous
import functools

import jax
import jax.numpy as jnp
from jax import lax
from jax.experimental import pallas as pl
from jax.experimental.pallas import tpu as pltpu
from jax.experimental.pallas import tpu_sc as plsc

N = 10000
E = 320000
D = 128
NC = 2
NS = 16
NW = NC * NS
EPT = E // NW
CH = 80
NCH = EPT // CH
SB = 25
NSUP = NCH // SB
OWN = 624
TAIL = N - OWN * NS
BR = 1000

_HI = lax.Precision.DEFAULT


def _proj_body(p_ref, d_ref, wq_ref, wk_ref, wv_ref, e1_ref, e2_ref,
               wv_out, hs_out, hd_out):
    p = p_ref[...]
    d = d_ref[...]
    wv_out[...] = jax.lax.dot_general(
        p, wv_ref[...], (((1,), (1,)), ((), ())), precision=_HI)
    w_k = jax.lax.dot_general(
        p, wk_ref[...], (((1,), (1,)), ((), ())),
        precision=_HI).astype(jnp.bfloat16).astype(jnp.float32)
    w_q = jax.lax.dot_general(
        d, wq_ref[...], (((1,), (1,)), ((), ())),
        precision=_HI).astype(jnp.bfloat16).astype(jnp.float32)

    def _matvec(a, v):
        vh = v.astype(jnp.bfloat16).astype(jnp.float32)
        return jax.lax.dot_general(a, vh, (((1,), (1,)), ((), ())),
                                   precision=_HI)

    hs_out[...] = _matvec(w_k, e1_ref[...])
    hd_out[...] = _matvec(w_q, e2_ref[...])


def _proj(p_u, d_u, Wq, Wk, Wv, e1, e2):
    full = pl.BlockSpec((D, D), lambda i: (0, 0))
    vec = pl.BlockSpec((1, D), lambda i: (0, 0))
    rows = pl.BlockSpec((BR, D), lambda i: (i, 0))
    col = pl.BlockSpec((BR, 1), lambda i: (i, 0))
    return pl.pallas_call(
        _proj_body,
        grid=(N // BR,),
        in_specs=[rows, rows, full, full, full, vec, vec],
        out_specs=[rows, col, col],
        out_shape=[
            jax.ShapeDtypeStruct((N, D), jnp.float32),
            jax.ShapeDtypeStruct((N, 1), jnp.float32),
            jax.ShapeDtypeStruct((N, 1), jnp.float32),
        ],
    )(p_u, d_u, Wq, Wk, Wv, e1, e2)


def _recip_body(d_ref, r_ref):
    r_ref[...] = 1.0 / (d_ref[0:1, :] + d_ref[1:2, :])


def _recip(den_p):
    return pl.pallas_call(
        _recip_body,
        out_shape=jax.ShapeDtypeStruct((1, N), jnp.float32),
    )(den_p)


@functools.cache
def _sc_kernels():
    mesh = plsc.VectorSubcoreMesh(core_axis_name="c", subcore_axis_name="s",
                                  num_cores=NC, num_subcores=NS)

    @functools.partial(
        pl.kernel,
        out_type=(
            jax.ShapeDtypeStruct((NW, NCH, CH), jnp.float32),
            jax.ShapeDtypeStruct((NC, N), jnp.float32),
        ),
        mesh=mesh,
        compiler_params=pltpu.CompilerParams(needs_layout_passes=False),
        scratch_types=[
            pltpu.VMEM((N,), jnp.float32),
            pltpu.VMEM((N,), jnp.float32),
            pltpu.VMEM((NCH, CH), jnp.int32),
            pltpu.VMEM((NCH, CH), jnp.int32),
            pltpu.VMEM((NCH, CH), jnp.float32),
            pltpu.VMEM((2000,), jnp.float32),
            pltpu.VMEM_SHARED((N,), jnp.float32),
        ],
    )
    def edge_kernel(hs_hbm, hd_hbm, src_hbm, dst_hbm, c_hbm, den_hbm,
                    hs_v, hd_v, src_v, dst_v, c_v, z_v, den_sh):
        cid = lax.axis_index("c")
        sid = lax.axis_index("s")
        wid = sid * NC + cid

        pltpu.sync_copy(hs_hbm, hs_v)
        pltpu.sync_copy(hd_hbm, hd_v)
        pltpu.sync_copy(src_hbm.at[wid], src_v)
        pltpu.sync_copy(dst_hbm.at[wid], dst_v)

        @pl.when(sid == 0)
        def _():
            def zz(i, carry):
                z_v[pl.ds(i * 16, 16)] = jnp.zeros((16,), jnp.float32)
                return carry
            lax.fori_loop(0, 2000 // 16, zz, 0)
            for k in range(N // 2000):
                pltpu.sync_copy(z_v, den_sh.at[pl.ds(k * 2000, 2000)])

        plsc.subcore_barrier()

        def chunk(j, c1):
            def inner(i, c2):
                sl = pl.ds(i * 16, 16)
                s16 = src_v[j, sl]
                d16 = dst_v[j, sl]
                cv = (plsc.load_gather(hs_v, [s16])
                      + plsc.load_gather(hd_v, [d16]))
                c_v[j, sl] = cv
                return c2
            lax.fori_loop(0, CH // 16, inner, 0)
            pltpu.sync_copy(c_v.at[j], den_sh.at[dst_v.at[j]], add=True)
            return c1
        lax.fori_loop(0, NCH, chunk, 0)

        pltpu.sync_copy(c_v, c_hbm.at[wid])
        plsc.subcore_barrier()

        @pl.when(sid == 0)
        def _():
            pltpu.sync_copy(den_sh, den_hbm.at[cid])

    @functools.partial(
        pl.kernel,
        out_type=jax.ShapeDtypeStruct((NC, N, D), jnp.float32),
        mesh=mesh,
        compiler_params=pltpu.CompilerParams(needs_layout_passes=False),
        scratch_types=[
            pltpu.VMEM((N,), jnp.float32),
            pltpu.VMEM((SB, CH), jnp.int32),
            pltpu.VMEM((SB, CH), jnp.int32),
            pltpu.VMEM((SB, CH), jnp.float32),
            pltpu.VMEM((CH, D), jnp.float32),
            pltpu.VMEM_SHARED((N, D), jnp.float32),
            pltpu.SemaphoreType.DMA,
        ],
    )
    def agg_kernel(wv_hbm, r_hbm, c_hbm, src_hbm, dst_hbm, hp_hbm,
                   r_v, src_v, dst_v, c_v, rowbuf, h_sh, sem):
        cid = lax.axis_index("c")
        sid = lax.axis_index("s")
        wid = sid * NC + cid

        pltpu.sync_copy(r_hbm, r_v)

        def zr(r, carry):
            for q in range(D // 16):
                rowbuf[r, pl.ds(q * 16, 16)] = jnp.zeros((16,), jnp.float32)
            return carry
        lax.fori_loop(0, CH, zr, 0)
        base = sid * OWN
        for k in range(OWN // CH):
            pltpu.sync_copy(rowbuf, h_sh.at[pl.ds(base + k * CH, CH)])
        rem = OWN % CH
        if rem:
            pltpu.sync_copy(rowbuf.at[pl.ds(0, rem)],
                            h_sh.at[pl.ds(base + (OWN // CH) * CH, rem)])

        @pl.when(sid == NS - 1)
        def _():
            pltpu.sync_copy(rowbuf.at[pl.ds(0, TAIL)],
                            h_sh.at[pl.ds(OWN * NS, TAIL)])

        plsc.subcore_barrier()

        def sup(g, carry):
            sup_i = wid * NSUP + g
            pltpu.sync_copy(src_hbm.at[sup_i], src_v)
            pltpu.sync_copy(dst_hbm.at[sup_i], dst_v)
            pltpu.sync_copy(c_hbm.at[sup_i], c_v)

            def chunk(j, c1):
                cp = pltpu.async_copy(wv_hbm.at[src_v.at[j]], rowbuf, sem)

                def cn(i, c2):
                    sl = pl.ds(i * 16, 16)
                    r16 = plsc.load_gather(r_v, [dst_v[j, sl]])
                    c_v[j, sl] = c_v[j, sl] * r16
                    return c2
                lax.fori_loop(0, CH // 16, cn, 0)
                cp.wait()

                def scale(i, c2):
                    cn16 = c_v[j, pl.ds(i * 16, 16)]
                    for r2 in range(16):
                        row = i * 16 + r2
                        cnr = cn16[r2]
                        for q in range(D // 16):
                            sl = pl.ds(q * 16, 16)
                            rowbuf[row, sl] = rowbuf[row, sl] * cnr
                    return c2
                lax.fori_loop(0, CH // 16, scale, 0)
                pltpu.sync_copy(rowbuf, h_sh.at[dst_v.at[j]], add=True)
                return c1
            lax.fori_loop(0, SB, chunk, 0)
            return carry
        lax.fori_loop(0, NSUP, sup, 0)

        plsc.subcore_barrier()
        pltpu.sync_copy(h_sh.at[pl.ds(base, OWN)],
                        hp_hbm.at[cid, pl.ds(base, OWN)])

        @pl.when(sid == NS - 1)
        def _():
            pltpu.sync_copy(h_sh.at[pl.ds(OWN * NS, TAIL)],
                            hp_hbm.at[cid, pl.ds(OWN * NS, TAIL)])

    return edge_kernel, agg_kernel


def _comb_body(a_ref, b_ref, o_ref):
    o_ref[...] = a_ref[...] + b_ref[...]


def _combine(a, b):
    rows = pl.BlockSpec((BR, D), lambda i: (i, 0))
    return pl.pallas_call(
        _comb_body,
        grid=(N // BR,),
        in_specs=[rows, rows],
        out_specs=rows,
        out_shape=jax.ShapeDtypeStruct((N, D), jnp.float32),
    )(a, b)


def kernel(d_u, p_u, edge_index, Wq, Wk, Wv, e1, e2):
    src = edge_index[0].reshape(NW, NCH, CH)
    dst = edge_index[1].reshape(NW, NCH, CH)
    w_v, hs, hd = _proj(p_u, d_u, Wq, Wk, Wv, e1, e2)
    edge_k, agg_k = _sc_kernels()
    c, den = edge_k(hs.reshape(N), hd.reshape(N), src, dst)
    r = _recip(den).reshape(N)
    hp = agg_k(w_v, r, c.reshape(NW * NSUP, SB, CH),
               src.reshape(NW * NSUP, SB, CH),
               dst.reshape(NW * NSUP, SB, CH))
    return _combine(hp[0], hp[1])


# speedup vs baseline: 31.3970x; 1.3335x over previous
import functools

import jax
import jax.numpy as jnp
from jax import lax
from jax.experimental import pallas as pl
from jax.experimental.pallas import tpu as pltpu
from jax.experimental.pallas import tpu_sc as plsc

N = 10000
E = 320000
D = 128
NC = 2
NS = 16
NW = NC * NS
EPT = E // NW
CH = 80
NCH = EPT // CH
SB = 25
NSUP = NCH // SB
OWN = 624
TAIL = N - OWN * NS
BR = 1000

_HI = lax.Precision.DEFAULT


def _proj_body(p_ref, d_ref, wq_ref, wk_ref, wv_ref, e1_ref, e2_ref,
               wv_out, hs_out, hd_out):
    p = p_ref[...]
    d = d_ref[...]
    wv_out[...] = jax.lax.dot_general(
        p, wv_ref[...], (((1,), (1,)), ((), ())), precision=_HI)
    w_k = jax.lax.dot_general(
        p, wk_ref[...], (((1,), (1,)), ((), ())),
        precision=_HI).astype(jnp.bfloat16).astype(jnp.float32)
    w_q = jax.lax.dot_general(
        d, wq_ref[...], (((1,), (1,)), ((), ())),
        precision=_HI).astype(jnp.bfloat16).astype(jnp.float32)

    def _matvec(a, v):
        vh = v.astype(jnp.bfloat16).astype(jnp.float32)
        return jax.lax.dot_general(a, vh, (((1,), (1,)), ((), ())),
                                   precision=_HI)

    hs_out[...] = _matvec(w_k, e1_ref[...])
    hd_out[...] = _matvec(w_q, e2_ref[...])


def _proj(p_u, d_u, Wq, Wk, Wv, e1, e2):
    full = pl.BlockSpec((D, D), lambda i: (0, 0))
    vec = pl.BlockSpec((1, D), lambda i: (0, 0))
    rows = pl.BlockSpec((BR, D), lambda i: (i, 0))
    col = pl.BlockSpec((BR, 1), lambda i: (i, 0))
    return pl.pallas_call(
        _proj_body,
        grid=(N // BR,),
        in_specs=[rows, rows, full, full, full, vec, vec],
        out_specs=[rows, col, col],
        out_shape=[
            jax.ShapeDtypeStruct((N, D), jnp.float32),
            jax.ShapeDtypeStruct((N, 1), jnp.float32),
            jax.ShapeDtypeStruct((N, 1), jnp.float32),
        ],
    )(p_u, d_u, Wq, Wk, Wv, e1, e2)


@functools.cache
def _sc_kernels():
    mesh = plsc.VectorSubcoreMesh(core_axis_name="c", subcore_axis_name="s",
                                  num_cores=NC, num_subcores=NS)

    @functools.partial(
        pl.kernel,
        out_type=(
            jax.ShapeDtypeStruct((NW, NCH, CH), jnp.float32),
            jax.ShapeDtypeStruct((NC, N), jnp.float32),
        ),
        mesh=mesh,
        compiler_params=pltpu.CompilerParams(needs_layout_passes=False),
        scratch_types=[
            pltpu.VMEM((N,), jnp.float32),
            pltpu.VMEM((N,), jnp.float32),
            pltpu.VMEM((NCH, CH), jnp.int32),
            pltpu.VMEM((NCH, CH), jnp.int32),
            pltpu.VMEM((NCH, CH), jnp.float32),
            pltpu.VMEM((2000,), jnp.float32),
            pltpu.VMEM_SHARED((N,), jnp.float32),
        ],
    )
    def edge_kernel(hs_hbm, hd_hbm, src_hbm, dst_hbm, c_hbm, den_hbm,
                    hs_v, hd_v, src_v, dst_v, c_v, z_v, den_sh):
        cid = lax.axis_index("c")
        sid = lax.axis_index("s")
        wid = sid * NC + cid

        pltpu.sync_copy(hs_hbm, hs_v)
        pltpu.sync_copy(hd_hbm, hd_v)
        pltpu.sync_copy(src_hbm.at[wid], src_v)
        pltpu.sync_copy(dst_hbm.at[wid], dst_v)

        @pl.when(sid == 0)
        def _():
            def zz(i, carry):
                z_v[pl.ds(i * 16, 16)] = jnp.zeros((16,), jnp.float32)
                return carry
            lax.fori_loop(0, 2000 // 16, zz, 0)
            for k in range(N // 2000):
                pltpu.sync_copy(z_v, den_sh.at[pl.ds(k * 2000, 2000)])

        plsc.subcore_barrier()

        def chunk(j, c1):
            def inner(i, c2):
                sl = pl.ds(i * 16, 16)
                s16 = src_v[j, sl]
                d16 = dst_v[j, sl]
                cv = (plsc.load_gather(hs_v, [s16])
                      + plsc.load_gather(hd_v, [d16]))
                c_v[j, sl] = cv
                return c2
            lax.fori_loop(0, CH // 16, inner, 0)
            pltpu.sync_copy(c_v.at[j], den_sh.at[dst_v.at[j]], add=True)
            return c1
        lax.fori_loop(0, NCH, chunk, 0)

        pltpu.sync_copy(c_v, c_hbm.at[wid])
        plsc.subcore_barrier()

        @pl.when(sid == 0)
        def _():
            pltpu.sync_copy(den_sh, den_hbm.at[cid])

    @functools.partial(
        pl.kernel,
        out_type=jax.ShapeDtypeStruct((NW, NCH, CH), jnp.float32),
        mesh=mesh,
        compiler_params=pltpu.CompilerParams(needs_layout_passes=False),
        scratch_types=[
            pltpu.VMEM((N,), jnp.float32),
            pltpu.VMEM((N,), jnp.float32),
            pltpu.VMEM((NCH, CH), jnp.int32),
            pltpu.VMEM((NCH, CH), jnp.float32),
        ],
    )
    def norm_kernel(den_hbm, c_hbm, dst_hbm, cn_hbm,
                    den_v, den2_v, dst_v, c_v):
        cid = lax.axis_index("c")
        sid = lax.axis_index("s")
        wid = sid * NC + cid
        pltpu.sync_copy(den_hbm.at[0], den_v)
        pltpu.sync_copy(den_hbm.at[1], den2_v)
        pltpu.sync_copy(dst_hbm.at[wid], dst_v)
        pltpu.sync_copy(c_hbm.at[wid], c_v)

        def addden(i, carry):
            sl = pl.ds(i * 16, 16)
            den_v[sl] = den_v[sl] + den2_v[sl]
            return carry
        lax.fori_loop(0, N // 16, addden, 0)

        def chunk(j, c1):
            def inner(i, c2):
                sl = pl.ds(i * 16, 16)
                d16 = plsc.load_gather(den_v, [dst_v[j, sl]])
                c_v[j, sl] = c_v[j, sl] / d16
                return c2
            lax.fori_loop(0, CH // 16, inner, 0)
            return c1
        lax.fori_loop(0, NCH, chunk, 0)
        pltpu.sync_copy(c_v, cn_hbm.at[wid])

    @functools.partial(
        pl.kernel,
        out_type=jax.ShapeDtypeStruct((NC, N, D), jnp.float32),
        mesh=mesh,
        compiler_params=pltpu.CompilerParams(needs_layout_passes=False),
        scratch_types=[
            pltpu.VMEM((SB, CH), jnp.int32),
            pltpu.VMEM((SB, CH), jnp.int32),
            pltpu.VMEM((SB, CH), jnp.float32),
            pltpu.VMEM((CH, D), jnp.float32),
            pltpu.VMEM((CH, D), jnp.float32),
            pltpu.VMEM_SHARED((N, D), jnp.float32),
            pltpu.SemaphoreType.DMA,
            pltpu.SemaphoreType.DMA,
        ],
    )
    def agg_kernel(wv_hbm, cn_hbm, src_hbm, dst_hbm, hp_hbm,
                   src_v, dst_v, c_v, buf0, buf1, h_sh, sem0, sem1):
        cid = lax.axis_index("c")
        sid = lax.axis_index("s")
        wid = sid * NC + cid

        def zr(r, carry):
            for q in range(D // 16):
                buf0[r, pl.ds(q * 16, 16)] = jnp.zeros((16,), jnp.float32)
            return carry
        lax.fori_loop(0, CH, zr, 0)
        base = sid * OWN
        for k in range(OWN // CH):
            pltpu.sync_copy(buf0, h_sh.at[pl.ds(base + k * CH, CH)])
        rem = OWN % CH
        if rem:
            pltpu.sync_copy(buf0.at[pl.ds(0, rem)],
                            h_sh.at[pl.ds(base + (OWN // CH) * CH, rem)])

        @pl.when(sid == NS - 1)
        def _():
            pltpu.sync_copy(buf0.at[pl.ds(0, TAIL)],
                            h_sh.at[pl.ds(OWN * NS, TAIL)])

        plsc.subcore_barrier()

        def gather(j, buf, sem):
            return pltpu.async_copy(wv_hbm.at[src_v.at[j]], buf, sem)

        def wait(j, buf, sem):
            pltpu.make_async_copy(wv_hbm.at[src_v.at[j]], buf, sem).wait()

        def scale_scatter(j, buf):
            def scale(i, c2):
                cn16 = c_v[j, pl.ds(i * 16, 16)]
                for r2 in range(16):
                    row = i * 16 + r2
                    cnr = cn16[r2]
                    for q in range(D // 16):
                        sl = pl.ds(q * 16, 16)
                        buf[row, sl] = buf[row, sl] * cnr
                return c2
            lax.fori_loop(0, CH // 16, scale, 0)
            pltpu.sync_copy(buf, h_sh.at[dst_v.at[j]], add=True)

        def sup(g, carry):
            sup_i = wid * NSUP + g
            pltpu.sync_copy(src_hbm.at[sup_i], src_v)
            pltpu.sync_copy(dst_hbm.at[sup_i], dst_v)
            pltpu.sync_copy(cn_hbm.at[sup_i], c_v)

            gather(0, buf0, sem0)

            def pair(t, c1):
                j0 = 2 * t
                gather(j0 + 1, buf1, sem1)
                wait(j0, buf0, sem0)
                scale_scatter(j0, buf0)
                gather(j0 + 2, buf0, sem0)
                wait(j0 + 1, buf1, sem1)
                scale_scatter(j0 + 1, buf1)
                return c1
            lax.fori_loop(0, SB // 2, pair, 0)

            wait(SB - 1, buf0, sem0)
            scale_scatter(SB - 1, buf0)
            return carry
        lax.fori_loop(0, NSUP, sup, 0)

        plsc.subcore_barrier()
        pltpu.sync_copy(h_sh.at[pl.ds(base, OWN)],
                        hp_hbm.at[cid, pl.ds(base, OWN)])

        @pl.when(sid == NS - 1)
        def _():
            pltpu.sync_copy(h_sh.at[pl.ds(OWN * NS, TAIL)],
                            hp_hbm.at[cid, pl.ds(OWN * NS, TAIL)])

    return edge_kernel, norm_kernel, agg_kernel


def _comb_body(a_ref, b_ref, o_ref):
    o_ref[...] = a_ref[...] + b_ref[...]


def _combine(a, b):
    rows = pl.BlockSpec((BR, D), lambda i: (i, 0))
    return pl.pallas_call(
        _comb_body,
        grid=(N // BR,),
        in_specs=[rows, rows],
        out_specs=rows,
        out_shape=jax.ShapeDtypeStruct((N, D), jnp.float32),
    )(a, b)


def kernel(d_u, p_u, edge_index, Wq, Wk, Wv, e1, e2):
    src = edge_index[0].reshape(NW, NCH, CH)
    dst = edge_index[1].reshape(NW, NCH, CH)
    w_v, hs, hd = _proj(p_u, d_u, Wq, Wk, Wv, e1, e2)
    edge_k, norm_k, agg_k = _sc_kernels()
    c, den = edge_k(hs.reshape(N), hd.reshape(N), src, dst)
    cn = norm_k(den, c, dst)
    hp = agg_k(w_v, cn.reshape(NW * NSUP, SB, CH),
               src.reshape(NW * NSUP, SB, CH),
               dst.reshape(NW * NSUP, SB, CH))
    return _combine(hp[0], hp[1])
